# baseline (speedup 1.0000x reference)
import functools

import jax
import jax.numpy as jnp
from jax import lax
from jax.experimental import pallas as pl
from jax.experimental.pallas import tpu as pltpu
from jax.experimental.pallas import tpu_sc as plsc

_NUM_CLASSES = 10
_NUM_ATTR = _NUM_CLASSES + 4
_NUM_PRIORS = 64 * 64 + 32 * 32 + 16 * 16
_NEGPOS_RATIO = 3
_BATCH = 32
_L = 16
_HW = ((64, 64), (32, 32), (16, 16))


def _dense_body(f0_ref, f1_ref, f2_ref, c0_ref, c1_ref, c2_ref,
                l0_ref, l1_ref, l2_ref,
                o0_ref, o1_ref, o2_ref, part_ref):
    b = pl.program_id(0)

    pos_cnt = jnp.float32(0.0)
    pos_sum = jnp.float32(0.0)
    loc_sum = jnp.float32(0.0)
    for x_ref, c_ref, l_ref, o_ref, (hh, ww) in zip(
            (f0_ref, f1_ref, f2_ref), (c0_ref, c1_ref, c2_ref),
            (l0_ref, l1_ref, l2_ref), (o0_ref, o1_ref, o2_ref), _HW):
        x = x_ref[0]
        cc = c_ref[0].astype(jnp.int32)
        lc = l_ref[0]
        conf = x[4:]
        m = jnp.max(conf, axis=0)
        s = jnp.sum(jnp.exp(conf - m[None]), axis=0)
        lse = m + jnp.log(s)
        katt = lax.broadcasted_iota(jnp.int32, (_NUM_CLASSES, hh, ww), 0)
        gathered = jnp.max(
            jnp.where(katt == cc[None], conf, -jnp.inf), axis=0)
        loss_c_all = lse - gathered
        pos = cc > 0
        posf = pos.astype(jnp.float32)
        o_ref[0] = jnp.where(pos, 0.0, loss_c_all)
        pos_cnt += jnp.sum(posf)
        pos_sum += jnp.sum(loss_c_all * posf)
        d = x[:4] - lc
        ad = jnp.abs(d)
        sl1 = jnp.where(ad < 1.0, 0.5 * d * d, ad - 0.5)
        loc_sum += jnp.sum(sl1 * posf[None])

    sub = lax.broadcasted_iota(jnp.int32, (8, 128), 0)
    lane = lax.broadcasted_iota(jnp.int32, (8, 128), 1)
    vals = jnp.where(lane == 4 * b, pos_cnt,
                     jnp.where(lane == 4 * b + 1, pos_sum,
                               jnp.where(lane == 4 * b + 2, loc_sum, 0.0)))
    vals = jnp.where(sub == 0, vals, 0.0)
    mask = (sub == 0) & (lane >= 4 * b) & (lane <= 4 * b + 2)

    @pl.when(b == 0)
    def _():
        part_ref[...] = vals

    @pl.when(b > 0)
    def _():
        part_ref[...] = jnp.where(mask, vals, part_ref[...])


def _dense(f0, f1, f2, c0, c1, c2, l0, l1, l2):
    specs_f = [pl.BlockSpec((1, _NUM_ATTR, h, w), lambda b: (b, 0, 0, 0))
               for h, w in _HW]
    specs_c = [pl.BlockSpec((1, h, w), lambda b: (b, 0, 0)) for h, w in _HW]
    specs_l = [pl.BlockSpec((1, 4, h, w), lambda b: (b, 0, 0, 0))
               for h, w in _HW]
    return pl.pallas_call(
        _dense_body,
        grid=(_BATCH,),
        in_specs=specs_f + specs_c + specs_l,
        out_specs=specs_c + [pl.BlockSpec((8, 128), lambda b: (0, 0))],
        out_shape=[jax.ShapeDtypeStruct((_BATCH, h, w), jnp.float32)
                   for h, w in _HW]
        + [jax.ShapeDtypeStruct((8, 128), jnp.float32)],
    )(f0, f1, f2, c0, c1, c2, l0, l1, l2)


def _sc_topk(loss0, loss1, loss2, partials):
    mesh = plsc.VectorSubcoreMesh(core_axis_name="c", subcore_axis_name="s")

    @functools.partial(
        pl.kernel,
        mesh=mesh,
        out_type=jax.ShapeDtypeStruct((_BATCH, _L), jnp.float32),
        scratch_types=[
            pltpu.VMEM((64, 64), jnp.float32),
            pltpu.VMEM((32, 32), jnp.float32),
            pltpu.VMEM((16, 16), jnp.float32),
            pltpu.VMEM((8, 128), jnp.float32),
            pltpu.VMEM((_L,), jnp.float32),
            pltpu.VMEM((_L * 257,), jnp.int32),
            pltpu.SemaphoreType.DMA,
        ],
        compiler_params=pltpu.CompilerParams(needs_layout_passes=False),
    )
    def body(l0_hbm, l1_hbm, l2_hbm, part_hbm, out_hbm,
             v0, v1, v2, part_v, res_v, hcl, sem):
        cid = lax.axis_index("c")
        sid = lax.axis_index("s")
        w = sid * 2 + cid
        c0 = pltpu.async_copy(l0_hbm.at[w], v0, sem)
        c1 = pltpu.async_copy(l1_hbm.at[w], v1, sem)
        c2 = pltpu.async_copy(l2_hbm.at[w], v2, sem)
        c3 = pltpu.async_copy(part_hbm, part_v, sem)
        c0.wait()
        c1.wait()
        c2.wait()
        c3.wait()

        lane = lax.iota(jnp.int32, 16)
        w_div = w // 4
        r1 = (w % 4) * 4 + 1
        r2 = (w % 4) * 4 + 2
        cntvec = jnp.zeros((_L,), jnp.float32)
        ownvec = jnp.zeros((_L,), jnp.float32)
        for j in range(8):
            v = part_v[0, j * _L:(j + 1) * _L]
            cntvec = cntvec + jnp.where(lane % 4 == 0, v, 0.0)
            ownvec = ownvec + jnp.where(
                (w_div == j) & ((lane == r1) | (lane == r2)), v, 0.0)
        num_pos_f = jnp.sum(cntvec)
        own = jnp.sum(ownvec)
        num_pos = num_pos_f.astype(jnp.int32)
        num_neg = jnp.minimum(_NEGPOS_RATIO * num_pos, _NUM_PRIORS - num_pos)

        ones = jnp.full((_L,), 1, jnp.int32)
        zc = jnp.zeros((_L,), jnp.int32)
        lane256 = lane * 257
        prefix = jnp.int32(0)
        k_lvl = num_neg
        for lvl, (sh, wbits) in enumerate(((23, 8), (15, 8), (7, 8), (0, 7))):
            for i in range(257):
                hcl[pl.ds(i * _L, _L)] = zc

            bmask = jnp.int32((1 << wbits) - 1)
            pfx = prefix

            for ref, (hh, ww) in zip((v0, v1, v2), _HW):
                ncol = ww // _L

                def row4(r0, carry, ref=ref, ncol=ncol, sh=sh, lvl=lvl,
                         bmask=bmask, wbits=wbits, pfx=pfx):
                    for rr in range(4):
                        r = r0 * 4 + rr
                        for u in range(ncol):
                            v = ref[r, u * _L:(u + 1) * _L]
                            b = lax.bitcast_convert_type(v, jnp.int32)
                            bk = lane256 + ((b >> sh) & bmask)
                            if lvl == 0:
                                plsc.addupdate_scatter(hcl, [bk], ones)
                            else:
                                m = (b >> (sh + wbits)) == pfx
                                plsc.addupdate_scatter(hcl, [bk], ones, mask=m)
                    return carry

                lax.fori_loop(0, hh // 4, row4, jnp.int32(0))

            s0c = jnp.int32(0)
            bsel_v = zc
            c_ab_v = zc
            for i in range(15, -1, -1):
                cblk = hcl[pl.ds(i * _L, _L)]
                for l in range(1, _L):
                    cblk = cblk + hcl[pl.ds(l * 257 + i * _L, _L)]
                rc = jnp.flip(cblk)
                cumc = jnp.cumsum(rc)
                prev_c = s0c + cumc - rc
                crossed = (prev_c < k_lvl) & (prev_c + rc >= k_lvl)
                bucket_id = jnp.int32(i * _L + 15) - lane
                bsel_v = bsel_v + jnp.where(crossed, bucket_id, 0)
                c_ab_v = c_ab_v + jnp.where(crossed, prev_c, 0)
                s0c = s0c + jnp.sum(cblk)
            bsel = jnp.sum(bsel_v)
            c_ab = jnp.sum(c_ab_v)
            prefix = (prefix << wbits) | bsel
            k_lvl = k_lvl - c_ab

        t_key = prefix
        gts_v = jnp.zeros((_L,), jnp.float32)
        gtc_v = zc
        for ref, (hh, ww) in zip((v0, v1, v2), _HW):
            ncol = ww // _L

            def frow4(r0, carry, ref=ref, ncol=ncol):
                gs, gc = carry
                for rr in range(4):
                    r = r0 * 4 + rr
                    for u in range(ncol):
                        v = ref[r, u * _L:(u + 1) * _L]
                        b = lax.bitcast_convert_type(v, jnp.int32)
                        gt = b > t_key
                        gs = gs + jnp.where(gt, v, 0.0)
                        gc = gc + jnp.where(gt, 1, 0)
                return gs, gc

            gts_v, gtc_v = lax.fori_loop(0, hh // 4, frow4, (gts_v, gtc_v))
        sum_gt = jnp.sum(gts_v)
        cnt_gt = jnp.sum(gtc_v)

        t_vec = lax.bitcast_convert_type(
            jnp.full((_L,), t_key, jnp.int32), jnp.float32)
        t_val = jnp.sum(jnp.where(lane == 0, t_vec, 0.0))
        neg_sum = sum_gt + (num_neg - cnt_gt).astype(jnp.float32) * t_val
        neg_sum = jnp.where(num_neg > 0, neg_sum, 0.0)

        total_w = own + neg_sum
        res_v[...] = jnp.where(lane == 0, total_w,
                               jnp.where(lane == 1, num_pos_f, 0.0))
        pltpu.sync_copy(res_v, out_hbm.at[w])

    return body(loss0, loss1, loss2, partials)


def kernel(feat0, feat1, feat2, loc_t, conf_t):
    b = feat0.shape[0]
    ci = conf_t.astype(jnp.int8)
    sizes = (4096, 1024, 256)
    offs = (0, 4096, 5120)
    confs = [ci[:, o:o + n].reshape(b, h, w)
             for (o, n, (h, w)) in zip(offs, sizes, _HW)]
    locs = [loc_t[:, o:o + n, :].transpose(0, 2, 1).reshape(b, 4, h, w)
            for (o, n, (h, w)) in zip(offs, sizes, _HW)]

    loss0, loss1, loss2, partials = _dense(feat0, feat1, feat2,
                                           *confs, *locs)
    out = _sc_topk(loss0, loss1, loss2, partials)
    num_pos_f = out[0, 1]
    denom = jnp.maximum(num_pos_f, 1.0)
    return jnp.sum(out[:, 0]) / denom


# speedup vs baseline: 1.1566x; 1.1566x over previous
import functools

import jax
import jax.numpy as jnp
from jax import lax
from jax.experimental import pallas as pl
from jax.experimental.pallas import tpu as pltpu
from jax.experimental.pallas import tpu_sc as plsc

_NUM_CLASSES = 10
_NUM_ATTR = _NUM_CLASSES + 4
_NUM_PRIORS = 64 * 64 + 32 * 32 + 16 * 16
_NEGPOS_RATIO = 3
_BATCH = 32
_L = 16
_HW = ((64, 64), (32, 32), (16, 16))


_IPP = 2


def _dense_body(f0_ref, f1_ref, f2_ref, c0_ref, c1_ref, c2_ref,
                l0_ref, l1_ref, l2_ref,
                o0_ref, o1_ref, o2_ref, part_ref):
    pid = pl.program_id(0)
    sub = lax.broadcasted_iota(jnp.int32, (8, 128), 0)
    lane = lax.broadcasted_iota(jnp.int32, (8, 128), 1)
    vals = jnp.zeros((8, 128), jnp.float32)
    mask = sub < 0

    for img in range(_IPP):
        b = pid * _IPP + img
        pos_cnt = jnp.float32(0.0)
        pos_sum = jnp.float32(0.0)
        loc_sum = jnp.float32(0.0)
        for x_ref, c_ref, l_ref, o_ref, (hh, ww) in zip(
                (f0_ref, f1_ref, f2_ref), (c0_ref, c1_ref, c2_ref),
                (l0_ref, l1_ref, l2_ref), (o0_ref, o1_ref, o2_ref), _HW):
            x = x_ref[img]
            cc = c_ref[img]
            lc = l_ref[img]
            conf = x[4:]
            m = jnp.max(conf, axis=0)
            s = jnp.sum(jnp.exp(conf - m[None]), axis=0)
            lse = m + jnp.log(s)
            katt = lax.broadcasted_iota(jnp.int32, (_NUM_CLASSES, hh, ww), 0)
            gathered = jnp.max(
                jnp.where(katt == cc[None], conf, -jnp.inf), axis=0)
            loss_c_all = lse - gathered
            pos = cc > 0
            posf = pos.astype(jnp.float32)
            o_ref[img] = jnp.where(pos, 0.0, loss_c_all)
            pos_cnt += jnp.sum(posf)
            pos_sum += jnp.sum(loss_c_all * posf)
            d = x[:4] - lc
            ad = jnp.abs(d)
            sl1 = jnp.where(ad < 1.0, 0.5 * d * d, ad - 0.5)
            loc_sum += jnp.sum(sl1 * posf[None])

        v = jnp.where(lane == 4 * b, pos_cnt,
                      jnp.where(lane == 4 * b + 1, pos_sum,
                                jnp.where(lane == 4 * b + 2, loc_sum, 0.0)))
        vals = vals + jnp.where(sub == 0, v, 0.0)
        mask = mask | ((sub == 0) & (lane >= 4 * b) & (lane <= 4 * b + 2))

    @pl.when(pid == 0)
    def _():
        part_ref[...] = vals

    @pl.when(pid > 0)
    def _():
        part_ref[...] = jnp.where(mask, vals, part_ref[...])


def _dense(f0, f1, f2, c0, c1, c2, l0, l1, l2):
    specs_f = [pl.BlockSpec((_IPP, _NUM_ATTR, h, w), lambda b: (b, 0, 0, 0))
               for h, w in _HW]
    specs_c = [pl.BlockSpec((_IPP, h, w), lambda b: (b, 0, 0))
               for h, w in _HW]
    specs_l = [pl.BlockSpec((_IPP, 4, h, w), lambda b: (b, 0, 0, 0))
               for h, w in _HW]
    return pl.pallas_call(
        _dense_body,
        grid=(_BATCH // _IPP,),
        in_specs=specs_f + specs_c + specs_l,
        out_specs=specs_c + [pl.BlockSpec((8, 128), lambda b: (0, 0))],
        out_shape=[jax.ShapeDtypeStruct((_BATCH, h, w), jnp.float32)
                   for h, w in _HW]
        + [jax.ShapeDtypeStruct((8, 128), jnp.float32)],
    )(f0, f1, f2, c0, c1, c2, l0, l1, l2)


def _sc_topk(loss0, loss1, loss2, partials):
    mesh = plsc.VectorSubcoreMesh(core_axis_name="c", subcore_axis_name="s")

    @functools.partial(
        pl.kernel,
        mesh=mesh,
        out_type=jax.ShapeDtypeStruct((_BATCH, _L), jnp.float32),
        scratch_types=[
            pltpu.VMEM((64, 64), jnp.float32),
            pltpu.VMEM((32, 32), jnp.float32),
            pltpu.VMEM((16, 16), jnp.float32),
            pltpu.VMEM((8, 128), jnp.float32),
            pltpu.VMEM((_L,), jnp.float32),
            pltpu.VMEM((_L * 257,), jnp.int32),
            pltpu.SemaphoreType.DMA,
        ],
        compiler_params=pltpu.CompilerParams(needs_layout_passes=False),
    )
    def body(l0_hbm, l1_hbm, l2_hbm, part_hbm, out_hbm,
             v0, v1, v2, part_v, res_v, hcl, sem):
        cid = lax.axis_index("c")
        sid = lax.axis_index("s")
        w = sid * 2 + cid
        c0 = pltpu.async_copy(l0_hbm.at[w], v0, sem)
        c1 = pltpu.async_copy(l1_hbm.at[w], v1, sem)
        c2 = pltpu.async_copy(l2_hbm.at[w], v2, sem)
        c3 = pltpu.async_copy(part_hbm, part_v, sem)
        c0.wait()
        c1.wait()
        c2.wait()
        c3.wait()

        lane = lax.iota(jnp.int32, 16)
        w_div = w // 4
        r1 = (w % 4) * 4 + 1
        r2 = (w % 4) * 4 + 2
        cntvec = jnp.zeros((_L,), jnp.float32)
        ownvec = jnp.zeros((_L,), jnp.float32)
        for j in range(8):
            v = part_v[0, j * _L:(j + 1) * _L]
            cntvec = cntvec + jnp.where(lane % 4 == 0, v, 0.0)
            ownvec = ownvec + jnp.where(
                (w_div == j) & ((lane == r1) | (lane == r2)), v, 0.0)
        num_pos_f = jnp.sum(cntvec)
        own = jnp.sum(ownvec)
        num_pos = num_pos_f.astype(jnp.int32)
        num_neg = jnp.minimum(_NEGPOS_RATIO * num_pos, _NUM_PRIORS - num_pos)

        ones = jnp.full((_L,), 1, jnp.int32)
        zc = jnp.zeros((_L,), jnp.int32)
        lane256 = lane * 257
        prefix = jnp.int32(0)
        k_lvl = num_neg
        for lvl, (sh, wbits) in enumerate(((23, 8), (15, 8), (7, 8), (0, 7))):
            for i in range(257):
                hcl[pl.ds(i * _L, _L)] = zc

            bmask = jnp.int32((1 << wbits) - 1)
            pfx = prefix

            for ref, (hh, ww) in zip((v0, v1, v2), _HW):
                ncol = ww // _L

                def row4(r0, carry, ref=ref, ncol=ncol, sh=sh, lvl=lvl,
                         bmask=bmask, wbits=wbits, pfx=pfx):
                    for rr in range(4):
                        r = r0 * 4 + rr
                        for u in range(ncol):
                            v = ref[r, u * _L:(u + 1) * _L]
                            b = lax.bitcast_convert_type(v, jnp.int32)
                            bk = lane256 + ((b >> sh) & bmask)
                            if lvl == 0:
                                plsc.addupdate_scatter(hcl, [bk], ones)
                            else:
                                m = (b >> (sh + wbits)) == pfx
                                plsc.addupdate_scatter(hcl, [bk], ones, mask=m)
                    return carry

                lax.fori_loop(0, hh // 4, row4, jnp.int32(0))

            s0c = jnp.int32(0)
            bsel_v = zc
            c_ab_v = zc
            for i in range(15, -1, -1):
                cblk = hcl[pl.ds(i * _L, _L)]
                for l in range(1, _L):
                    cblk = cblk + hcl[pl.ds(l * 257 + i * _L, _L)]
                rc = jnp.flip(cblk)
                cumc = jnp.cumsum(rc)
                prev_c = s0c + cumc - rc
                crossed = (prev_c < k_lvl) & (prev_c + rc >= k_lvl)
                bucket_id = jnp.int32(i * _L + 15) - lane
                bsel_v = bsel_v + jnp.where(crossed, bucket_id, 0)
                c_ab_v = c_ab_v + jnp.where(crossed, prev_c, 0)
                s0c = s0c + jnp.sum(cblk)
            bsel = jnp.sum(bsel_v)
            c_ab = jnp.sum(c_ab_v)
            prefix = (prefix << wbits) | bsel
            k_lvl = k_lvl - c_ab

        t_key = prefix
        gts_v = jnp.zeros((_L,), jnp.float32)
        gtc_v = zc
        for ref, (hh, ww) in zip((v0, v1, v2), _HW):
            ncol = ww // _L

            def frow4(r0, carry, ref=ref, ncol=ncol):
                gs, gc = carry
                for rr in range(4):
                    r = r0 * 4 + rr
                    for u in range(ncol):
                        v = ref[r, u * _L:(u + 1) * _L]
                        b = lax.bitcast_convert_type(v, jnp.int32)
                        gt = b > t_key
                        gs = gs + jnp.where(gt, v, 0.0)
                        gc = gc + jnp.where(gt, 1, 0)
                return gs, gc

            gts_v, gtc_v = lax.fori_loop(0, hh // 4, frow4, (gts_v, gtc_v))
        sum_gt = jnp.sum(gts_v)
        cnt_gt = jnp.sum(gtc_v)

        t_vec = lax.bitcast_convert_type(
            jnp.full((_L,), t_key, jnp.int32), jnp.float32)
        t_val = jnp.sum(jnp.where(lane == 0, t_vec, 0.0))
        neg_sum = sum_gt + (num_neg - cnt_gt).astype(jnp.float32) * t_val
        neg_sum = jnp.where(num_neg > 0, neg_sum, 0.0)

        total_w = own + neg_sum
        res_v[...] = jnp.where(lane == 0, total_w,
                               jnp.where(lane == 1, num_pos_f, 0.0))
        pltpu.sync_copy(res_v, out_hbm.at[w])

    return body(loss0, loss1, loss2, partials)


def kernel(feat0, feat1, feat2, loc_t, conf_t):
    b = feat0.shape[0]
    ci = conf_t.astype(jnp.int32)
    sizes = (4096, 1024, 256)
    offs = (0, 4096, 5120)
    confs = [ci[:, o:o + n].reshape(b, h, w)
             for (o, n, (h, w)) in zip(offs, sizes, _HW)]
    locs = [loc_t[:, o:o + n, :].transpose(0, 2, 1).reshape(b, 4, h, w)
            for (o, n, (h, w)) in zip(offs, sizes, _HW)]

    loss0, loss1, loss2, partials = _dense(feat0, feat1, feat2,
                                           *confs, *locs)
    out = _sc_topk(loss0, loss1, loss2, partials)
    num_pos_f = out[0, 1]
    denom = jnp.maximum(num_pos_f, 1.0)
    return jnp.sum(out[:, 0]) / denom


# speedup vs baseline: 1.2176x; 1.0527x over previous
import functools

import jax
import jax.numpy as jnp
from jax import lax
from jax.experimental import pallas as pl
from jax.experimental.pallas import tpu as pltpu
from jax.experimental.pallas import tpu_sc as plsc

_NUM_CLASSES = 10
_NUM_ATTR = _NUM_CLASSES + 4
_NUM_PRIORS = 64 * 64 + 32 * 32 + 16 * 16
_NEGPOS_RATIO = 3
_BATCH = 32
_L = 16
_HW = ((64, 64), (32, 32), (16, 16))


_IPP = 4


def _dense_body(f0_ref, f1_ref, f2_ref, c0_ref, c1_ref, c2_ref,
                l0_ref, l1_ref, l2_ref,
                o0_ref, o1_ref, o2_ref, part_ref):
    pid = pl.program_id(0)
    sub = lax.broadcasted_iota(jnp.int32, (8, 128), 0)
    lane = lax.broadcasted_iota(jnp.int32, (8, 128), 1)
    vals = jnp.zeros((8, 128), jnp.float32)
    mask = sub < 0

    for img in range(_IPP):
        b = pid * _IPP + img
        pos_cnt = jnp.float32(0.0)
        pos_sum = jnp.float32(0.0)
        loc_sum = jnp.float32(0.0)
        for x_ref, c_ref, l_ref, o_ref, (hh, ww) in zip(
                (f0_ref, f1_ref, f2_ref), (c0_ref, c1_ref, c2_ref),
                (l0_ref, l1_ref, l2_ref), (o0_ref, o1_ref, o2_ref), _HW):
            x = x_ref[img]
            cc = c_ref[img]
            lc = l_ref[img]
            conf = x[4:]
            m = jnp.max(conf, axis=0)
            s = jnp.sum(jnp.exp(conf - m[None]), axis=0)
            lse = m + jnp.log(s)
            katt = lax.broadcasted_iota(jnp.int32, (_NUM_CLASSES, hh, ww), 0)
            gathered = jnp.max(
                jnp.where(katt == cc[None], conf, -jnp.inf), axis=0)
            loss_c_all = lse - gathered
            pos = cc > 0
            posf = pos.astype(jnp.float32)
            o_ref[img] = jnp.where(pos, 0.0, loss_c_all)
            pos_cnt += jnp.sum(posf)
            pos_sum += jnp.sum(loss_c_all * posf)
            d = x[:4] - lc
            ad = jnp.abs(d)
            sl1 = jnp.where(ad < 1.0, 0.5 * d * d, ad - 0.5)
            loc_sum += jnp.sum(sl1 * posf[None])

        v = jnp.where(lane == 4 * b, pos_cnt,
                      jnp.where(lane == 4 * b + 1, pos_sum,
                                jnp.where(lane == 4 * b + 2, loc_sum, 0.0)))
        vals = vals + jnp.where(sub == 0, v, 0.0)
        mask = mask | ((sub == 0) & (lane >= 4 * b) & (lane <= 4 * b + 2))

    @pl.when(pid == 0)
    def _():
        part_ref[...] = vals

    @pl.when(pid > 0)
    def _():
        part_ref[...] = jnp.where(mask, vals, part_ref[...])


def _dense(f0, f1, f2, c0, c1, c2, l0, l1, l2):
    specs_f = [pl.BlockSpec((_IPP, _NUM_ATTR, h, w), lambda b: (b, 0, 0, 0))
               for h, w in _HW]
    specs_c = [pl.BlockSpec((_IPP, h, w), lambda b: (b, 0, 0))
               for h, w in _HW]
    specs_l = [pl.BlockSpec((_IPP, 4, h, w), lambda b: (b, 0, 0, 0))
               for h, w in _HW]
    return pl.pallas_call(
        _dense_body,
        grid=(_BATCH // _IPP,),
        in_specs=specs_f + specs_c + specs_l,
        out_specs=specs_c + [pl.BlockSpec((8, 128), lambda b: (0, 0))],
        out_shape=[jax.ShapeDtypeStruct((_BATCH, h, w), jnp.float32)
                   for h, w in _HW]
        + [jax.ShapeDtypeStruct((8, 128), jnp.float32)],
    )(f0, f1, f2, c0, c1, c2, l0, l1, l2)


def _sc_topk(loss0, loss1, loss2, partials):
    mesh = plsc.VectorSubcoreMesh(core_axis_name="c", subcore_axis_name="s")

    @functools.partial(
        pl.kernel,
        mesh=mesh,
        out_type=jax.ShapeDtypeStruct((_BATCH, _L), jnp.float32),
        scratch_types=[
            pltpu.VMEM((64, 64), jnp.float32),
            pltpu.VMEM((32, 32), jnp.float32),
            pltpu.VMEM((16, 16), jnp.float32),
            pltpu.VMEM((8, 128), jnp.float32),
            pltpu.VMEM((_L,), jnp.float32),
            pltpu.VMEM((_L * 257,), jnp.int32),
            pltpu.SemaphoreType.DMA,
        ],
        compiler_params=pltpu.CompilerParams(needs_layout_passes=False),
    )
    def body(l0_hbm, l1_hbm, l2_hbm, part_hbm, out_hbm,
             v0, v1, v2, part_v, res_v, hcl, sem):
        cid = lax.axis_index("c")
        sid = lax.axis_index("s")
        w = sid * 2 + cid
        c0 = pltpu.async_copy(l0_hbm.at[w], v0, sem)
        c1 = pltpu.async_copy(l1_hbm.at[w], v1, sem)
        c2 = pltpu.async_copy(l2_hbm.at[w], v2, sem)
        c3 = pltpu.async_copy(part_hbm, part_v, sem)
        c0.wait()
        c1.wait()
        c2.wait()
        c3.wait()

        lane = lax.iota(jnp.int32, 16)
        w_div = w // 4
        r1 = (w % 4) * 4 + 1
        r2 = (w % 4) * 4 + 2
        cntvec = jnp.zeros((_L,), jnp.float32)
        ownvec = jnp.zeros((_L,), jnp.float32)
        for j in range(8):
            v = part_v[0, j * _L:(j + 1) * _L]
            cntvec = cntvec + jnp.where(lane % 4 == 0, v, 0.0)
            ownvec = ownvec + jnp.where(
                (w_div == j) & ((lane == r1) | (lane == r2)), v, 0.0)
        num_pos_f = jnp.sum(cntvec)
        own = jnp.sum(ownvec)
        num_pos = num_pos_f.astype(jnp.int32)
        num_neg = jnp.minimum(_NEGPOS_RATIO * num_pos, _NUM_PRIORS - num_pos)

        ones = jnp.full((_L,), 1, jnp.int32)
        zc = jnp.zeros((_L,), jnp.int32)
        lane256 = lane * 257
        prefix = jnp.int32(0)
        k_lvl = num_neg
        for lvl, (sh, wbits) in enumerate(((23, 8), (15, 8), (7, 8), (0, 7))):
            for i in range(257):
                hcl[pl.ds(i * _L, _L)] = zc

            bmask = jnp.int32((1 << wbits) - 1)
            pfx = prefix

            for ref, (hh, ww) in zip((v0, v1, v2), _HW):
                ncol = ww // _L

                def row4(r0, carry, ref=ref, ncol=ncol, sh=sh, lvl=lvl,
                         bmask=bmask, wbits=wbits, pfx=pfx):
                    for rr in range(4):
                        r = r0 * 4 + rr
                        for u in range(ncol):
                            v = ref[r, u * _L:(u + 1) * _L]
                            b = lax.bitcast_convert_type(v, jnp.int32)
                            bk = lane256 + ((b >> sh) & bmask)
                            if lvl == 0:
                                plsc.addupdate_scatter(hcl, [bk], ones)
                            else:
                                m = (b >> (sh + wbits)) == pfx
                                plsc.addupdate_scatter(hcl, [bk], ones, mask=m)
                    return carry

                lax.fori_loop(0, hh // 4, row4, jnp.int32(0))

            s0c = jnp.int32(0)
            bsel_v = zc
            c_ab_v = zc
            for i in range(15, -1, -1):
                cblk = hcl[pl.ds(i * _L, _L)]
                for l in range(1, _L):
                    cblk = cblk + hcl[pl.ds(l * 257 + i * _L, _L)]
                rc = jnp.flip(cblk)
                cumc = jnp.cumsum(rc)
                prev_c = s0c + cumc - rc
                crossed = (prev_c < k_lvl) & (prev_c + rc >= k_lvl)
                bucket_id = jnp.int32(i * _L + 15) - lane
                bsel_v = bsel_v + jnp.where(crossed, bucket_id, 0)
                c_ab_v = c_ab_v + jnp.where(crossed, prev_c, 0)
                s0c = s0c + jnp.sum(cblk)
            bsel = jnp.sum(bsel_v)
            c_ab = jnp.sum(c_ab_v)
            prefix = (prefix << wbits) | bsel
            k_lvl = k_lvl - c_ab

        t_key = prefix
        gts_v = jnp.zeros((_L,), jnp.float32)
        gtc_v = zc
        for ref, (hh, ww) in zip((v0, v1, v2), _HW):
            ncol = ww // _L

            def frow4(r0, carry, ref=ref, ncol=ncol):
                gs, gc = carry
                for rr in range(4):
                    r = r0 * 4 + rr
                    for u in range(ncol):
                        v = ref[r, u * _L:(u + 1) * _L]
                        b = lax.bitcast_convert_type(v, jnp.int32)
                        gt = b > t_key
                        gs = gs + jnp.where(gt, v, 0.0)
                        gc = gc + jnp.where(gt, 1, 0)
                return gs, gc

            gts_v, gtc_v = lax.fori_loop(0, hh // 4, frow4, (gts_v, gtc_v))
        sum_gt = jnp.sum(gts_v)
        cnt_gt = jnp.sum(gtc_v)

        t_vec = lax.bitcast_convert_type(
            jnp.full((_L,), t_key, jnp.int32), jnp.float32)
        t_val = jnp.sum(jnp.where(lane == 0, t_vec, 0.0))
        neg_sum = sum_gt + (num_neg - cnt_gt).astype(jnp.float32) * t_val
        neg_sum = jnp.where(num_neg > 0, neg_sum, 0.0)

        total_w = own + neg_sum
        res_v[...] = jnp.where(lane == 0, total_w,
                               jnp.where(lane == 1, num_pos_f, 0.0))
        pltpu.sync_copy(res_v, out_hbm.at[w])

    return body(loss0, loss1, loss2, partials)


def kernel(feat0, feat1, feat2, loc_t, conf_t):
    b = feat0.shape[0]
    ci = conf_t.astype(jnp.int32)
    sizes = (4096, 1024, 256)
    offs = (0, 4096, 5120)
    confs = [ci[:, o:o + n].reshape(b, h, w)
             for (o, n, (h, w)) in zip(offs, sizes, _HW)]
    locs = [loc_t[:, o:o + n, :].transpose(0, 2, 1).reshape(b, 4, h, w)
            for (o, n, (h, w)) in zip(offs, sizes, _HW)]

    loss0, loss1, loss2, partials = _dense(feat0, feat1, feat2,
                                           *confs, *locs)
    out = _sc_topk(loss0, loss1, loss2, partials)
    num_pos_f = out[0, 1]
    denom = jnp.maximum(num_pos_f, 1.0)
    return jnp.sum(out[:, 0]) / denom


# speedup vs baseline: 1.2347x; 1.0141x over previous
import functools

import jax
import jax.numpy as jnp
from jax import lax
from jax.experimental import pallas as pl
from jax.experimental.pallas import tpu as pltpu
from jax.experimental.pallas import tpu_sc as plsc

_NUM_CLASSES = 10
_NUM_ATTR = _NUM_CLASSES + 4
_NUM_PRIORS = 64 * 64 + 32 * 32 + 16 * 16
_NEGPOS_RATIO = 3
_BATCH = 32
_L = 16
_HW = ((64, 64), (32, 32), (16, 16))


_IPP = 8


def _dense_body(f0_ref, f1_ref, f2_ref, c0_ref, c1_ref, c2_ref,
                l0_ref, l1_ref, l2_ref,
                o0_ref, o1_ref, o2_ref, part_ref):
    pid = pl.program_id(0)
    sub = lax.broadcasted_iota(jnp.int32, (8, 128), 0)
    lane = lax.broadcasted_iota(jnp.int32, (8, 128), 1)
    vals = jnp.zeros((8, 128), jnp.float32)
    mask = sub < 0

    for img in range(_IPP):
        b = pid * _IPP + img
        pos_cnt = jnp.float32(0.0)
        pos_sum = jnp.float32(0.0)
        loc_sum = jnp.float32(0.0)
        for x_ref, c_ref, l_ref, o_ref, (hh, ww) in zip(
                (f0_ref, f1_ref, f2_ref), (c0_ref, c1_ref, c2_ref),
                (l0_ref, l1_ref, l2_ref), (o0_ref, o1_ref, o2_ref), _HW):
            x = x_ref[img]
            cc = c_ref[img]
            lc = l_ref[img]
            conf = x[4:]
            m = jnp.max(conf, axis=0)
            s = jnp.sum(jnp.exp(conf - m[None]), axis=0)
            lse = m + jnp.log(s)
            katt = lax.broadcasted_iota(jnp.int32, (_NUM_CLASSES, hh, ww), 0)
            gathered = jnp.max(
                jnp.where(katt == cc[None], conf, -jnp.inf), axis=0)
            loss_c_all = lse - gathered
            pos = cc > 0
            posf = pos.astype(jnp.float32)
            o_ref[img] = jnp.where(pos, 0.0, loss_c_all)
            pos_cnt += jnp.sum(posf)
            pos_sum += jnp.sum(loss_c_all * posf)
            d = x[:4] - lc
            ad = jnp.abs(d)
            sl1 = jnp.where(ad < 1.0, 0.5 * d * d, ad - 0.5)
            loc_sum += jnp.sum(sl1 * posf[None])

        v = jnp.where(lane == 4 * b, pos_cnt,
                      jnp.where(lane == 4 * b + 1, pos_sum,
                                jnp.where(lane == 4 * b + 2, loc_sum, 0.0)))
        vals = vals + jnp.where(sub == 0, v, 0.0)
        mask = mask | ((sub == 0) & (lane >= 4 * b) & (lane <= 4 * b + 2))

    @pl.when(pid == 0)
    def _():
        part_ref[...] = vals

    @pl.when(pid > 0)
    def _():
        part_ref[...] = jnp.where(mask, vals, part_ref[...])


def _dense(f0, f1, f2, c0, c1, c2, l0, l1, l2):
    specs_f = [pl.BlockSpec((_IPP, _NUM_ATTR, h, w), lambda b: (b, 0, 0, 0))
               for h, w in _HW]
    specs_c = [pl.BlockSpec((_IPP, h, w), lambda b: (b, 0, 0))
               for h, w in _HW]
    specs_l = [pl.BlockSpec((_IPP, 4, h, w), lambda b: (b, 0, 0, 0))
               for h, w in _HW]
    return pl.pallas_call(
        _dense_body,
        grid=(_BATCH // _IPP,),
        in_specs=specs_f + specs_c + specs_l,
        out_specs=specs_c + [pl.BlockSpec((8, 128), lambda b: (0, 0))],
        out_shape=[jax.ShapeDtypeStruct((_BATCH, h, w), jnp.float32)
                   for h, w in _HW]
        + [jax.ShapeDtypeStruct((8, 128), jnp.float32)],
    )(f0, f1, f2, c0, c1, c2, l0, l1, l2)


def _sc_topk(loss0, loss1, loss2, partials):
    mesh = plsc.VectorSubcoreMesh(core_axis_name="c", subcore_axis_name="s")

    @functools.partial(
        pl.kernel,
        mesh=mesh,
        out_type=jax.ShapeDtypeStruct((_BATCH, _L), jnp.float32),
        scratch_types=[
            pltpu.VMEM((64, 64), jnp.float32),
            pltpu.VMEM((32, 32), jnp.float32),
            pltpu.VMEM((16, 16), jnp.float32),
            pltpu.VMEM((8, 128), jnp.float32),
            pltpu.VMEM((_L,), jnp.float32),
            pltpu.VMEM((_L * 257,), jnp.int32),
            pltpu.SemaphoreType.DMA,
        ],
        compiler_params=pltpu.CompilerParams(needs_layout_passes=False),
    )
    def body(l0_hbm, l1_hbm, l2_hbm, part_hbm, out_hbm,
             v0, v1, v2, part_v, res_v, hcl, sem):
        cid = lax.axis_index("c")
        sid = lax.axis_index("s")
        w = sid * 2 + cid
        c0 = pltpu.async_copy(l0_hbm.at[w], v0, sem)
        c1 = pltpu.async_copy(l1_hbm.at[w], v1, sem)
        c2 = pltpu.async_copy(l2_hbm.at[w], v2, sem)
        c3 = pltpu.async_copy(part_hbm, part_v, sem)
        c0.wait()
        c1.wait()
        c2.wait()
        c3.wait()

        lane = lax.iota(jnp.int32, 16)
        w_div = w // 4
        r1 = (w % 4) * 4 + 1
        r2 = (w % 4) * 4 + 2
        cntvec = jnp.zeros((_L,), jnp.float32)
        ownvec = jnp.zeros((_L,), jnp.float32)
        for j in range(8):
            v = part_v[0, j * _L:(j + 1) * _L]
            cntvec = cntvec + jnp.where(lane % 4 == 0, v, 0.0)
            ownvec = ownvec + jnp.where(
                (w_div == j) & ((lane == r1) | (lane == r2)), v, 0.0)
        num_pos_f = jnp.sum(cntvec)
        own = jnp.sum(ownvec)
        num_pos = num_pos_f.astype(jnp.int32)
        num_neg = jnp.minimum(_NEGPOS_RATIO * num_pos, _NUM_PRIORS - num_pos)

        ones = jnp.full((_L,), 1, jnp.int32)
        zc = jnp.zeros((_L,), jnp.int32)
        lane256 = lane * 257
        prefix = jnp.int32(0)
        k_lvl = num_neg
        for lvl, (sh, wbits) in enumerate(((23, 8), (15, 8), (7, 8), (0, 7))):
            for i in range(257):
                hcl[pl.ds(i * _L, _L)] = zc

            bmask = jnp.int32((1 << wbits) - 1)
            pfx = prefix

            for ref, (hh, ww) in zip((v0, v1, v2), _HW):
                ncol = ww // _L

                def row4(r0, carry, ref=ref, ncol=ncol, sh=sh, lvl=lvl,
                         bmask=bmask, wbits=wbits, pfx=pfx):
                    for rr in range(4):
                        r = r0 * 4 + rr
                        for u in range(ncol):
                            v = ref[r, u * _L:(u + 1) * _L]
                            b = lax.bitcast_convert_type(v, jnp.int32)
                            bk = lane256 + ((b >> sh) & bmask)
                            if lvl == 0:
                                plsc.addupdate_scatter(hcl, [bk], ones)
                            else:
                                m = (b >> (sh + wbits)) == pfx
                                plsc.addupdate_scatter(hcl, [bk], ones, mask=m)
                    return carry

                lax.fori_loop(0, hh // 4, row4, jnp.int32(0))

            s0c = jnp.int32(0)
            bsel_v = zc
            c_ab_v = zc
            for i in range(15, -1, -1):
                cblk = hcl[pl.ds(i * _L, _L)]
                for l in range(1, _L):
                    cblk = cblk + hcl[pl.ds(l * 257 + i * _L, _L)]
                rc = jnp.flip(cblk)
                cumc = jnp.cumsum(rc)
                prev_c = s0c + cumc - rc
                crossed = (prev_c < k_lvl) & (prev_c + rc >= k_lvl)
                bucket_id = jnp.int32(i * _L + 15) - lane
                bsel_v = bsel_v + jnp.where(crossed, bucket_id, 0)
                c_ab_v = c_ab_v + jnp.where(crossed, prev_c, 0)
                s0c = s0c + jnp.sum(cblk)
            bsel = jnp.sum(bsel_v)
            c_ab = jnp.sum(c_ab_v)
            prefix = (prefix << wbits) | bsel
            k_lvl = k_lvl - c_ab

        t_key = prefix
        gts_v = jnp.zeros((_L,), jnp.float32)
        gtc_v = zc
        for ref, (hh, ww) in zip((v0, v1, v2), _HW):
            ncol = ww // _L

            def frow4(r0, carry, ref=ref, ncol=ncol):
                gs, gc = carry
                for rr in range(4):
                    r = r0 * 4 + rr
                    for u in range(ncol):
                        v = ref[r, u * _L:(u + 1) * _L]
                        b = lax.bitcast_convert_type(v, jnp.int32)
                        gt = b > t_key
                        gs = gs + jnp.where(gt, v, 0.0)
                        gc = gc + jnp.where(gt, 1, 0)
                return gs, gc

            gts_v, gtc_v = lax.fori_loop(0, hh // 4, frow4, (gts_v, gtc_v))
        sum_gt = jnp.sum(gts_v)
        cnt_gt = jnp.sum(gtc_v)

        t_vec = lax.bitcast_convert_type(
            jnp.full((_L,), t_key, jnp.int32), jnp.float32)
        t_val = jnp.sum(jnp.where(lane == 0, t_vec, 0.0))
        neg_sum = sum_gt + (num_neg - cnt_gt).astype(jnp.float32) * t_val
        neg_sum = jnp.where(num_neg > 0, neg_sum, 0.0)

        total_w = own + neg_sum
        res_v[...] = jnp.where(lane == 0, total_w,
                               jnp.where(lane == 1, num_pos_f, 0.0))
        pltpu.sync_copy(res_v, out_hbm.at[w])

    return body(loss0, loss1, loss2, partials)


def kernel(feat0, feat1, feat2, loc_t, conf_t):
    b = feat0.shape[0]
    ci = conf_t.astype(jnp.int32)
    sizes = (4096, 1024, 256)
    offs = (0, 4096, 5120)
    confs = [ci[:, o:o + n].reshape(b, h, w)
             for (o, n, (h, w)) in zip(offs, sizes, _HW)]
    locs = [loc_t[:, o:o + n, :].transpose(0, 2, 1).reshape(b, 4, h, w)
            for (o, n, (h, w)) in zip(offs, sizes, _HW)]

    loss0, loss1, loss2, partials = _dense(feat0, feat1, feat2,
                                           *confs, *locs)
    out = _sc_topk(loss0, loss1, loss2, partials)
    num_pos_f = out[0, 1]
    denom = jnp.maximum(num_pos_f, 1.0)
    return jnp.sum(out[:, 0]) / denom
